# baseline (speedup 1.0000x reference)
import functools

import jax
import jax.numpy as jnp
from jax import lax
from jax.experimental import pallas as pl
from jax.experimental.pallas import tpu as pltpu
from jax.experimental.pallas import tpu_sc as plsc

VOCAB_SIZE = 30522
SEQ_LEN = 512
HIDDEN = 1024
BATCH = 32

NUM_CORES = 2
NUM_SUBCORES = 16
NUM_WORKERS = NUM_CORES * NUM_SUBCORES
NUM_TOKENS = BATCH * SEQ_LEN

CHUNK_SEQS = (8, 12, 12)
GATHER_CHUNK = 64


@functools.lru_cache(maxsize=None)
def _make_sc_gather(seq_off, num_seqs):
  token_off = seq_off * SEQ_LEN
  tokens = num_seqs * SEQ_LEN
  rows_per_worker = tokens // NUM_WORKERS
  assert rows_per_worker % GATHER_CHUNK == 0

  mesh = plsc.VectorSubcoreMesh(core_axis_name="c", subcore_axis_name="s")

  @functools.partial(
      pl.kernel,
      mesh=mesh,
      out_type=jax.ShapeDtypeStruct((tokens, HIDDEN), jnp.float32),
      scratch_types=[
          pltpu.VMEM((rows_per_worker,), jnp.int32),
          pltpu.VMEM((GATHER_CHUNK, HIDDEN), jnp.float32),
          pltpu.SemaphoreType.DMA,
      ],
  )
  def sc_gather(table_hbm, idx_hbm, out_hbm, idx_v, rows_v, sem):
    wid = lax.axis_index("s") * NUM_CORES + lax.axis_index("c")
    base = wid * rows_per_worker
    pltpu.sync_copy(idx_hbm.at[pl.ds(token_off + base, rows_per_worker)],
                    idx_v)

    @pl.loop(0, rows_per_worker, step=GATHER_CHUNK)
    def _(c):
      pltpu.async_copy(
          table_hbm.at[idx_v.at[pl.ds(c, GATHER_CHUNK)]], rows_v, sem
      ).wait()
      pltpu.sync_copy(rows_v, out_hbm.at[pl.ds(base + c, GATHER_CHUNK)])

  return sc_gather


def _ln_body(g_ref, ttf_ref, pos_ref, tok_ref, gamma_ref, beta_ref, o_ref):
  g = g_ref[...]
  tt = ttf_ref[0, 0, :]
  t0 = tok_ref[0, :]
  t1 = tok_ref[1, :]
  x = g + pos_ref[...] + t0[None, :] + tt[:, None] * (t1 - t0)[None, :]
  mu = jnp.mean(x, axis=-1, keepdims=True)
  var = jnp.mean(jnp.square(x - mu), axis=-1, keepdims=True)
  y = (x - mu) * lax.rsqrt(var + 1e-12)
  o_ref[...] = y * gamma_ref[...][None, :] + beta_ref[...][None, :]


def _ln_body_alias(g_ref, ttf_ref, pos_ref, tok_ref, gamma_ref, beta_ref, _,
                   o_ref):
  _ln_body(g_ref, ttf_ref, pos_ref, tok_ref, gamma_ref, beta_ref, o_ref)


@functools.partial(jax.jit, static_argnums=(6, 7))
def _tc_layernorm_chunk(gathered, ttf, pos_table, tok_type_table, gamma, beta,
                        seq_off, num_seqs, out=None):
  first = out is None
  in_specs = [
      pl.BlockSpec((SEQ_LEN, HIDDEN), lambda b: (b, 0)),
      pl.BlockSpec((1, 1, SEQ_LEN), lambda b, s=seq_off: (s + b, 0, 0)),
      pl.BlockSpec((SEQ_LEN, HIDDEN), lambda b: (0, 0)),
      pl.BlockSpec((2, HIDDEN), lambda b: (0, 0)),
      pl.BlockSpec((HIDDEN,), lambda b: (0,)),
      pl.BlockSpec((HIDDEN,), lambda b: (0,)),
  ]
  args = [gathered, ttf, pos_table, tok_type_table, gamma, beta]
  if not first:
    in_specs.append(pl.BlockSpec(memory_space=pl.ANY))
    args.append(out)
  return pl.pallas_call(
      _ln_body if first else _ln_body_alias,
      grid=(num_seqs,),
      in_specs=in_specs,
      out_specs=pl.BlockSpec((SEQ_LEN, HIDDEN),
                             lambda b, s=seq_off: (s + b, 0)),
      out_shape=jax.ShapeDtypeStruct((NUM_TOKENS, HIDDEN), jnp.float32),
      input_output_aliases={} if first else {6: 0},
      compiler_params=pltpu.CompilerParams(
          dimension_semantics=("parallel",)),
  )(*args)


@jax.jit
def kernel(input_ids, token_type_ids, word_table, pos_table, tok_type_table,
           gamma, beta):
  idx = input_ids.reshape(NUM_TOKENS).astype(jnp.int32)
  ttf = token_type_ids.astype(jnp.float32).reshape(BATCH, 1, SEQ_LEN)

  gathered = []
  seq_off = 0
  for ns in CHUNK_SEQS:
    gathered.append(_make_sc_gather(seq_off, ns)(word_table, idx))
    seq_off += ns

  out = None
  seq_off = 0
  for ns, g in zip(CHUNK_SEQS, gathered):
    out = _tc_layernorm_chunk(g, ttf, pos_table, tok_type_table, gamma, beta,
                              seq_off, ns, out)
    seq_off += ns
  return out.reshape(BATCH, SEQ_LEN, HIDDEN)


# speedup vs baseline: 1.0173x; 1.0173x over previous
import functools

import jax
import jax.numpy as jnp
from jax import lax
from jax.experimental import pallas as pl
from jax.experimental.pallas import tpu as pltpu
from jax.experimental.pallas import tpu_sc as plsc

VOCAB_SIZE = 30522
SEQ_LEN = 512
HIDDEN = 1024
BATCH = 32

NUM_CORES = 2
NUM_SUBCORES = 16
NUM_WORKERS = NUM_CORES * NUM_SUBCORES
NUM_TOKENS = BATCH * SEQ_LEN

CHUNK_SEQS = (8, 12, 12)
GATHER_CHUNK = 32


@functools.lru_cache(maxsize=None)
def _make_sc_gather(seq_off, num_seqs):
  token_off = seq_off * SEQ_LEN
  tokens = num_seqs * SEQ_LEN
  rows_per_worker = tokens // NUM_WORKERS
  num_steps = rows_per_worker // GATHER_CHUNK
  assert num_steps % 2 == 0 and num_steps > 0

  mesh = plsc.VectorSubcoreMesh(core_axis_name="c", subcore_axis_name="s")

  @functools.partial(
      pl.kernel,
      mesh=mesh,
      out_type=jax.ShapeDtypeStruct((tokens, HIDDEN), jnp.float32),
      scratch_types=[
          pltpu.VMEM((rows_per_worker,), jnp.int32),
          pltpu.VMEM((GATHER_CHUNK, HIDDEN), jnp.float32),
          pltpu.VMEM((GATHER_CHUNK, HIDDEN), jnp.float32),
          pltpu.SemaphoreType.DMA,
          pltpu.SemaphoreType.DMA,
          pltpu.SemaphoreType.DMA,
          pltpu.SemaphoreType.DMA,
      ],
  )
  def sc_gather(table_hbm, idx_hbm, out_hbm, idx_v, rows0, rows1,
                gsem0, gsem1, osem0, osem1):
    wid = lax.axis_index("s") * NUM_CORES + lax.axis_index("c")
    base = wid * rows_per_worker
    pltpu.sync_copy(idx_hbm.at[pl.ds(token_off + base, rows_per_worker)],
                    idx_v)

    bufs = (rows0, rows1)
    gsems = (gsem0, gsem1)
    osems = (osem0, osem1)

    def g_args(c, b):
      return (table_hbm.at[idx_v.at[pl.ds(c * GATHER_CHUNK, GATHER_CHUNK)]],
              bufs[b], gsems[b])

    def o_args(c, b):
      return (bufs[b],
              out_hbm.at[pl.ds(base + c * GATHER_CHUNK, GATHER_CHUNK)],
              osems[b])

    pltpu.async_copy(*g_args(0, 0))

    @pl.loop(0, num_steps, step=2)
    def _(c0):
      c1 = c0 + 1
      pltpu.make_async_copy(*g_args(c0, 0)).wait()

      @pl.when(c0 > 0)
      def _():
        pltpu.make_async_copy(*o_args(c0 - 1, 1)).wait()

      pltpu.async_copy(*g_args(c1, 1))
      pltpu.async_copy(*o_args(c0, 0))

      pltpu.make_async_copy(*g_args(c1, 1)).wait()
      pltpu.make_async_copy(*o_args(c0, 0)).wait()

      @pl.when(c1 + 1 < num_steps)
      def _():
        pltpu.async_copy(*g_args(c1 + 1, 0))

      pltpu.async_copy(*o_args(c1, 1))

    pltpu.make_async_copy(*o_args(num_steps - 1, 1)).wait()

  return sc_gather


def _ln_body(g_ref, ttf_ref, pos_ref, tok_ref, gamma_ref, beta_ref, o_ref):
  g = g_ref[...]
  tt = ttf_ref[0, 0, :]
  t0 = tok_ref[0, :]
  t1 = tok_ref[1, :]
  x = g + pos_ref[...] + t0[None, :] + tt[:, None] * (t1 - t0)[None, :]
  mu = jnp.mean(x, axis=-1, keepdims=True)
  var = jnp.mean(jnp.square(x - mu), axis=-1, keepdims=True)
  y = (x - mu) * lax.rsqrt(var + 1e-12)
  o_ref[...] = y * gamma_ref[...][None, :] + beta_ref[...][None, :]


def _ln_body_alias(g_ref, ttf_ref, pos_ref, tok_ref, gamma_ref, beta_ref, _,
                   o_ref):
  _ln_body(g_ref, ttf_ref, pos_ref, tok_ref, gamma_ref, beta_ref, o_ref)


@functools.partial(jax.jit, static_argnums=(6, 7))
def _tc_layernorm_chunk(gathered, ttf, pos_table, tok_type_table, gamma, beta,
                        seq_off, num_seqs, out=None):
  first = out is None
  in_specs = [
      pl.BlockSpec((SEQ_LEN, HIDDEN), lambda b: (b, 0)),
      pl.BlockSpec((1, 1, SEQ_LEN), lambda b, s=seq_off: (s + b, 0, 0)),
      pl.BlockSpec((SEQ_LEN, HIDDEN), lambda b: (0, 0)),
      pl.BlockSpec((2, HIDDEN), lambda b: (0, 0)),
      pl.BlockSpec((HIDDEN,), lambda b: (0,)),
      pl.BlockSpec((HIDDEN,), lambda b: (0,)),
  ]
  args = [gathered, ttf, pos_table, tok_type_table, gamma, beta]
  if not first:
    in_specs.append(pl.BlockSpec(memory_space=pl.ANY))
    args.append(out)
  return pl.pallas_call(
      _ln_body if first else _ln_body_alias,
      grid=(num_seqs,),
      in_specs=in_specs,
      out_specs=pl.BlockSpec((SEQ_LEN, HIDDEN),
                             lambda b, s=seq_off: (s + b, 0)),
      out_shape=jax.ShapeDtypeStruct((NUM_TOKENS, HIDDEN), jnp.float32),
      input_output_aliases={} if first else {6: 0},
      compiler_params=pltpu.CompilerParams(
          dimension_semantics=("parallel",)),
  )(*args)


@jax.jit
def kernel(input_ids, token_type_ids, word_table, pos_table, tok_type_table,
           gamma, beta):
  idx = input_ids.reshape(NUM_TOKENS).astype(jnp.int32)
  ttf = token_type_ids.astype(jnp.float32).reshape(BATCH, 1, SEQ_LEN)

  gathered = []
  seq_off = 0
  for ns in CHUNK_SEQS:
    gathered.append(_make_sc_gather(seq_off, ns)(word_table, idx))
    seq_off += ns

  out = None
  seq_off = 0
  for ns, g in zip(CHUNK_SEQS, gathered):
    out = _tc_layernorm_chunk(g, ttf, pos_table, tok_type_table, gamma, beta,
                              seq_off, ns, out)
    seq_off += ns
  return out.reshape(BATCH, SEQ_LEN, HIDDEN)
